# Initial kernel scaffold; baseline (speedup 1.0000x reference)
import jax
import jax.numpy as jnp
from jax.experimental import pallas as pl


def kernel(x, h, W1, b1, W2, b2, W3, b3, ln_g, ln_b, W4, b4, edge_indices, batch_size):
    raise NotImplementedError("write your pallas kernel here")


# speedup vs baseline: 7.1025x; 7.1025x over previous
import functools

import jax
import jax.numpy as jnp
from jax import lax
from jax.experimental import pallas as pl
from jax.experimental.pallas import tpu as pltpu
from jax.experimental.pallas import tpu_sc as plsc

NC = 2
NS = 16
L = 16
CH = 80
F = 32


def _row_partition(N):
  rpt = -(-N // (NS * 8)) * 8
  last = N - (NS - 1) * rpt
  assert last > 0
  return rpt, last


def _zero_vmem(buf, rows, cols=None):
  z = jnp.zeros((L,), jnp.float32)
  if cols is None:
    for i in range(rows // L):
      buf[pl.ds(i * L, L)] = z
  else:
    for r in range(rows):
      for q in range(cols // L):
        buf[r, pl.ds(q * L, L)] = z


def _fill_slab(zbuf, nbuf, dst, do, n):
  o = 0
  while o < n:
    m = min(nbuf, n - o)
    pltpu.sync_copy(zbuf.at[pl.ds(0, m)], dst.at[pl.ds(do + o, m)])
    o += m


def _drain_slab(src, so, dst, do, n, buf, nbuf):
  o = 0
  while o < n:
    m = min(nbuf, n - o)
    pltpu.sync_copy(src.at[pl.ds(so + o, m)], buf.at[pl.ds(0, m)])
    pltpu.sync_copy(buf.at[pl.ds(0, m)], dst.at[pl.ds(do + o, m)])
    o += m


def _tile_slab_init(zbuf, nbuf, sh, s, N):
  rpt, last = _row_partition(N)

  @pl.when(s < NS - 1)
  def _():
    _fill_slab(zbuf, nbuf, sh, s * rpt, rpt)

  @pl.when(s == NS - 1)
  def _():
    _fill_slab(zbuf, nbuf, sh, (NS - 1) * rpt, last)


def _tile_slab_drain(sh, out, out_off, s, N, buf, nbuf):
  rpt, last = _row_partition(N)

  @pl.when(s < NS - 1)
  def _():
    _drain_slab(sh, s * rpt, out, out_off + s * rpt, rpt, buf, nbuf)

  @pl.when(s == NS - 1)
  def _():
    _drain_slab(sh, (NS - 1) * rpt, out, out_off + (NS - 1) * rpt, last,
                buf, nbuf)


def _edge_dist_counts(E, N):
  EC = E // NC
  chunks_per_core = EC // CH
  tile_chunks = chunks_per_core // NS
  extra = chunks_per_core % NS
  mesh = plsc.VectorSubcoreMesh(core_axis_name="c", subcore_axis_name="s")

  @functools.partial(
      pl.kernel,
      compiler_params=pltpu.CompilerParams(use_tc_tiling_on_sc=False),
      out_type=[jax.ShapeDtypeStruct((NC * E, 8), jnp.float32),
                jax.ShapeDtypeStruct((NC * N,), jnp.float32)],
      mesh=mesh,
      scratch_types=[pltpu.VMEM((CH,), jnp.int32),
                     pltpu.VMEM((CH,), jnp.int32),
                     pltpu.VMEM((CH, 8), jnp.float32),
                     pltpu.VMEM((CH, 8), jnp.float32),
                     pltpu.VMEM((CH,), jnp.float32),
                     pltpu.VMEM((800,), jnp.float32),
                     pltpu.VMEM_SHARED((N,), jnp.float32),
                     pltpu.SemaphoreType.DMA,
                     pltpu.SemaphoreType.DMA])
  def body(x8, rows, cols, xg_out, cnt_out,
           ir, ic, xr, xc, ones, zb, cnt_sh, sem1, sem2):
    c = lax.axis_index("c")
    s = lax.axis_index("s")
    for g in range(CH // L):
      ones[pl.ds(g * L, L)] = jnp.full((L,), 1.0, jnp.float32)
    _zero_vmem(zb, 800)
    _tile_slab_init(zb, 800, cnt_sh, s, N)
    plsc.subcore_barrier()

    base_c = c * EC

    def chunk_body(i, carry):
      off = base_c + (i * NS + s) * CH
      pltpu.sync_copy(rows.at[pl.ds(off, CH)], ir)
      pltpu.sync_copy(cols.at[pl.ds(off, CH)], ic)
      cp1 = pltpu.async_copy(x8.at[ir], xr, sem1)
      cp2 = pltpu.async_copy(x8.at[ic], xc, sem2)
      cp1.wait()
      cp2.wait()
      pltpu.sync_copy(xr, xg_out.at[pl.ds(off, CH)])
      pltpu.sync_copy(xc, xg_out.at[pl.ds(E + off, CH)])
      pltpu.sync_copy(ones, cnt_sh.at[ir], add=True)
      return carry

    trips = tile_chunks + jnp.where(s < extra, 1, 0)
    lax.fori_loop(0, trips, chunk_body, 0)
    plsc.subcore_barrier()
    _tile_slab_drain(cnt_sh, cnt_out, c * N, s, N, zb, 800)

  return body


def _pick_block(n, cap):
  for b in range(min(cap, n) // 8 * 8, 7, -8):
    if n % b == 0:
      return b
  return n


def _filt_kernel(E, H, BE=None):
  BE = BE or _pick_block(E, 4096)

  def body(xr_ref, xc_ref, w1_ref, b1_ref, w2_ref, b2_ref, out_ref):
    dv = xr_ref[0] - xc_ref[0]
    d2 = jnp.sum(dv * dv, axis=1, keepdims=True)
    dist = jnp.sqrt(d2)
    hid = dist * w1_ref[...] + b1_ref[...]
    hid = hid * jax.nn.sigmoid(hid)
    f = jnp.dot(hid, w2_ref[...],
                preferred_element_type=jnp.float32) + b2_ref[...]
    out_ref[0] = f[:, :F]
    out_ref[1] = f[:, F:]

  return pl.pallas_call(
      body,
      grid=(E // BE,),
      in_specs=[
          pl.BlockSpec((1, BE, 8), lambda i: (0, i, 0)),
          pl.BlockSpec((1, BE, 8), lambda i: (1, i, 0)),
          pl.BlockSpec((1, H), lambda i: (0, 0)),
          pl.BlockSpec((1, H), lambda i: (0, 0)),
          pl.BlockSpec((H, 2 * F), lambda i: (0, 0)),
          pl.BlockSpec((1, 2 * F), lambda i: (0, 0)),
      ],
      out_specs=pl.BlockSpec((2, BE, F), lambda i: (0, i, 0)),
      out_shape=jax.ShapeDtypeStruct((2, E, F), jnp.float32),
  )


def _gather_mul_scatter(E, N):
  tile_chunks = E // CH // NS
  mesh = plsc.VectorSubcoreMesh(core_axis_name="c", subcore_axis_name="s")

  @functools.partial(
      pl.kernel,
      compiler_params=pltpu.CompilerParams(use_tc_tiling_on_sc=False),
      out_type=jax.ShapeDtypeStruct((NC * N, F), jnp.float32),
      mesh=mesh,
      scratch_types=[pltpu.VMEM((CH,), jnp.int32),
                     pltpu.VMEM((CH,), jnp.int32),
                     pltpu.VMEM((CH, F), jnp.float32),
                     pltpu.VMEM((CH, F), jnp.float32),
                     pltpu.VMEM((CH, F), jnp.float32),
                     pltpu.VMEM((256, F), jnp.float32),
                     pltpu.VMEM_SHARED((N, F), jnp.float32),
                     pltpu.SemaphoreType.DMA])
  def body(hsplit, rows, cols, fsplit, agg_out,
           ir, ic, hb, fb, mb, zb, agg_sh, sem):
    c = lax.axis_index("c")
    s = lax.axis_index("s")
    _zero_vmem(zb, 256, F)
    _tile_slab_init(zb, 256, agg_sh, s, N)
    plsc.subcore_barrier()
    cN = c * N
    cE = c * E

    def chunk_body(i, carry):
      off = (s * tile_chunks + i) * CH
      pltpu.sync_copy(rows.at[pl.ds(off, CH)], ir)
      pltpu.sync_copy(cols.at[pl.ds(off, CH)], ic)
      for g in range(CH // L):
        ic[pl.ds(g * L, L)] = ic[pl.ds(g * L, L)] + cN
      cp = pltpu.async_copy(hsplit.at[ic], hb, sem)
      pltpu.sync_copy(fsplit.at[pl.ds(cE + off, CH)], fb)
      cp.wait()
      for r in range(CH):
        for q in range(F // L):
          mb[r, pl.ds(q * L, L)] = (hb[r, pl.ds(q * L, L)]
                                    * fb[r, pl.ds(q * L, L)])
      pltpu.sync_copy(mb, agg_sh.at[ir], add=True)
      return carry

    lax.fori_loop(0, tile_chunks, chunk_body, 0)
    plsc.subcore_barrier()
    _tile_slab_drain(agg_sh, agg_out, cN, s, N, zb, 256)

  return body


def _update_kernel(N, D, H, BN=None):
  BN = BN or _pick_block(N, 1024)

  def body(h_ref, a0_ref, a1_ref, c0_ref, c1_ref,
           w3_ref, b3_ref, g_ref, bb_ref, w4_ref, b4_ref, out_ref):
    cnt = jnp.maximum(c0_ref[0] + c1_ref[0], 1.0)
    agg = jnp.concatenate([a0_ref[0], a1_ref[0]], axis=1) / cnt
    u = jnp.concatenate([h_ref[...], agg], axis=1)
    t = jnp.dot(u, w3_ref[...],
                preferred_element_type=jnp.float32) + b3_ref[...]
    m = jnp.mean(t, axis=1, keepdims=True)
    v = jnp.mean((t - m) ** 2, axis=1, keepdims=True)
    t = (t - m) / jnp.sqrt(v + 1e-5) * g_ref[...] + bb_ref[...]
    t = t * jax.nn.sigmoid(t)
    out_ref[...] = jnp.dot(t, w4_ref[...],
                           preferred_element_type=jnp.float32) + b4_ref[...]

  return pl.pallas_call(
      body,
      grid=(N // BN,),
      in_specs=[
          pl.BlockSpec((BN, D), lambda i: (i, 0)),
          pl.BlockSpec((1, BN, F), lambda i: (0, i, 0)),
          pl.BlockSpec((1, BN, F), lambda i: (1, i, 0)),
          pl.BlockSpec((1, BN, 1), lambda i: (0, i, 0)),
          pl.BlockSpec((1, BN, 1), lambda i: (1, i, 0)),
          pl.BlockSpec((2 * D, H), lambda i: (0, 0)),
          pl.BlockSpec((1, H), lambda i: (0, 0)),
          pl.BlockSpec((1, H), lambda i: (0, 0)),
          pl.BlockSpec((1, H), lambda i: (0, 0)),
          pl.BlockSpec((H, D), lambda i: (0, 0)),
          pl.BlockSpec((1, D), lambda i: (0, 0)),
      ],
      out_specs=pl.BlockSpec((BN, D), lambda i: (i, 0)),
      out_shape=jax.ShapeDtypeStruct((N, D), jnp.float32),
  )


def kernel(x, h, W1, b1, W2, b2, W3, b3, ln_g, ln_b, W4, b4,
           edge_indices, batch_size):
  B, N, D = h.shape
  E = edge_indices.shape[1]
  H = W1.shape[1]
  assert B == 1 and D == 2 * F
  assert E % (NC * CH) == 0 and (E // CH) % NS == 0
  assert N % 8 == 0

  x2 = x.reshape(N, 3)
  x8 = jnp.zeros((N, 8), jnp.float32).at[:, :3].set(x2)
  h2 = h.reshape(N, D)
  hsplit = jnp.concatenate([h2[:, :F], h2[:, F:]], axis=0)
  rows = edge_indices[0]
  cols = edge_indices[1]

  xg, cnts = _edge_dist_counts(E, N)(x8, rows, cols)
  fsplit = _filt_kernel(E, H)(xg.reshape(NC, E, 8), xg.reshape(NC, E, 8),
                              W1, b1.reshape(1, H),
                              W2, b2.reshape(1, 2 * F))
  aggflat = _gather_mul_scatter(E, N)(hsplit, rows, cols,
                                      fsplit.reshape(NC * E, F))
  agg2 = aggflat.reshape(NC, N, F)
  cnt2 = cnts.reshape(NC, N, 1)
  out = _update_kernel(N, D, H)(h2, agg2, agg2, cnt2, cnt2,
                                W3, b3.reshape(1, H), ln_g.reshape(1, H),
                                ln_b.reshape(1, H), W4, b4.reshape(1, D))
  return out.reshape(B, N, D)


# speedup vs baseline: 7.7960x; 1.0976x over previous
import functools

import jax
import jax.numpy as jnp
from jax import lax
from jax.experimental import pallas as pl
from jax.experimental.pallas import tpu as pltpu
from jax.experimental.pallas import tpu_sc as plsc

NC = 2
NS = 16
L = 16
CH = 80
F = 32


def _row_partition(N):
  rpt = -(-N // (NS * 8)) * 8
  last = N - (NS - 1) * rpt
  assert last > 0
  return rpt, last


def _zero_vmem(buf, rows, cols=None):
  z = jnp.zeros((L,), jnp.float32)
  if cols is None:
    for i in range(rows // L):
      buf[pl.ds(i * L, L)] = z
  else:
    for r in range(rows):
      for q in range(cols // L):
        buf[r, pl.ds(q * L, L)] = z


def _fill_slab(zbuf, nbuf, dst, do, n):
  o = 0
  while o < n:
    m = min(nbuf, n - o)
    pltpu.sync_copy(zbuf.at[pl.ds(0, m)], dst.at[pl.ds(do + o, m)])
    o += m


def _drain_slab(src, so, dst, do, n, buf, nbuf):
  o = 0
  while o < n:
    m = min(nbuf, n - o)
    pltpu.sync_copy(src.at[pl.ds(so + o, m)], buf.at[pl.ds(0, m)])
    pltpu.sync_copy(buf.at[pl.ds(0, m)], dst.at[pl.ds(do + o, m)])
    o += m


def _tile_slab_init(zbuf, nbuf, sh, s, N):
  rpt, last = _row_partition(N)

  @pl.when(s < NS - 1)
  def _():
    _fill_slab(zbuf, nbuf, sh, s * rpt, rpt)

  @pl.when(s == NS - 1)
  def _():
    _fill_slab(zbuf, nbuf, sh, (NS - 1) * rpt, last)


def _tile_slab_drain(sh, out, out_off, s, N, buf, nbuf):
  rpt, last = _row_partition(N)

  @pl.when(s < NS - 1)
  def _():
    _drain_slab(sh, s * rpt, out, out_off + s * rpt, rpt, buf, nbuf)

  @pl.when(s == NS - 1)
  def _():
    _drain_slab(sh, (NS - 1) * rpt, out, out_off + (NS - 1) * rpt, last,
                buf, nbuf)


def _edge_dist_counts(E, N):
  EC = E // NC
  U = EC // CH
  tile_chunks = U // NS
  extra = U % NS
  mesh = plsc.VectorSubcoreMesh(core_axis_name="c", subcore_axis_name="s")

  NB = 4
  scratch = []
  for _ in range(NB):
    scratch += [pltpu.VMEM((CH,), jnp.int32),
                pltpu.VMEM((CH,), jnp.int32),
                pltpu.VMEM((CH, 8), jnp.float32),
                pltpu.VMEM((CH, 8), jnp.float32)]
  scratch += [pltpu.VMEM((CH,), jnp.float32),
              pltpu.VMEM((800,), jnp.float32),
              pltpu.VMEM_SHARED((N,), jnp.float32)]
  scratch += [pltpu.SemaphoreType.DMA] * (4 * NB)

  @functools.partial(
      pl.kernel,
      compiler_params=pltpu.CompilerParams(use_tc_tiling_on_sc=False),
      out_type=[jax.ShapeDtypeStruct((NC * E, 8), jnp.float32),
                jax.ShapeDtypeStruct((NC * N,), jnp.float32)],
      mesh=mesh,
      scratch_types=scratch)
  def body(x8, rows, cols, xg_out, cnt_out, *sc):
    ir = [sc[4 * j + 0] for j in range(NB)]
    ic = [sc[4 * j + 1] for j in range(NB)]
    xr = [sc[4 * j + 2] for j in range(NB)]
    xc = [sc[4 * j + 3] for j in range(NB)]
    ones, zb, cnt_sh = sc[4 * NB:4 * NB + 3]
    sems = sc[4 * NB + 3:]
    ix = sems[0:NB]
    gs = sems[NB:2 * NB]
    ws = sems[2 * NB:3 * NB]
    ss = sems[3 * NB:4 * NB]

    c = lax.axis_index("c")
    s = lax.axis_index("s")
    for g in range(CH // L):
      ones[pl.ds(g * L, L)] = jnp.full((L,), 1.0, jnp.float32)
    _zero_vmem(zb, 800)
    _tile_slab_init(zb, 800, cnt_sh, s, N)
    plsc.subcore_barrier()

    base_c = c * EC
    n = tile_chunks + jnp.where(s < extra, 1, 0)

    def off_of(k):
      return base_c + (k * NS + s) * CH

    def fire_idx(k, j):
      o = off_of(k)
      pltpu.async_copy(rows.at[pl.ds(o, CH)], ir[j], ix[j])
      pltpu.async_copy(cols.at[pl.ds(o, CH)], ic[j], ix[j])

    def drain_idx(j):
      pltpu.make_async_copy(rows.at[pl.ds(0, CH)], ir[j], ix[j]).wait()
      pltpu.make_async_copy(cols.at[pl.ds(0, CH)], ic[j], ix[j]).wait()

    def fire_gather(j):
      pltpu.async_copy(x8.at[ir[j]], xr[j], gs[j])
      pltpu.async_copy(x8.at[ic[j]], xc[j], gs[j])

    def drain_gather(j):
      pltpu.make_async_copy(x8.at[ir[j]], xr[j], gs[j]).wait()
      pltpu.make_async_copy(x8.at[ic[j]], xc[j], gs[j]).wait()

    fire_idx(0, 0)
    fire_idx(1, 1)
    drain_idx(0)
    fire_gather(0)

    S = (tile_chunks + 12) // 4

    def super_body(i, carry):
      for j in range(4):
        k = i * 4 + j
        j1, j2 = (j + 1) % 4, (j + 2) % 4

        @pl.when(jnp.logical_and(k >= 2, k - 2 < n))
        def _():
          pltpu.make_async_copy(rows.at[pl.ds(0, CH)], ones, ss[j2]).wait()

        @pl.when(jnp.logical_and(k >= 3, k - 3 < n))
        def _():
          pltpu.make_async_copy(rows.at[pl.ds(0, CH)], xr[j1], ws[j1]).wait()
          pltpu.make_async_copy(rows.at[pl.ds(0, CH)], xc[j1], ws[j1]).wait()

        @pl.when(k + 1 < n)
        def _():
          drain_idx(j1)
          fire_gather(j1)

        @pl.when(k < n)
        def _():
          drain_gather(j)
          o = off_of(k)
          pltpu.async_copy(xr[j], xg_out.at[pl.ds(o, CH)], ws[j])
          pltpu.async_copy(xc[j], xg_out.at[pl.ds(E + o, CH)], ws[j])
          pltpu.async_copy(ones, cnt_sh.at[ir[j]], ss[j], add=True)

        @pl.when(k + 2 < n)
        def _():
          fire_idx(k + 2, j2)
      return carry

    lax.fori_loop(0, S, super_body, 0)
    plsc.subcore_barrier()
    _tile_slab_drain(cnt_sh, cnt_out, c * N, s, N, zb, 800)

  return body


def _pick_block(n, cap):
  for b in range(min(cap, n) // 8 * 8, 7, -8):
    if n % b == 0:
      return b
  return n


def _filt_kernel(E, H, BE=None):
  BE = BE or _pick_block(E, 4096)

  def body(xr_ref, xc_ref, w1_ref, b1_ref, w2_ref, b2_ref, out_ref):
    dv = xr_ref[0] - xc_ref[0]
    d2 = jnp.sum(dv * dv, axis=1, keepdims=True)
    dist = jnp.sqrt(d2)
    hid = dist * w1_ref[...] + b1_ref[...]
    hid = hid * jax.nn.sigmoid(hid)
    f = jnp.dot(hid, w2_ref[...],
                preferred_element_type=jnp.float32) + b2_ref[...]
    out_ref[0] = f[:, :F]
    out_ref[1] = f[:, F:]

  return pl.pallas_call(
      body,
      grid=(E // BE,),
      in_specs=[
          pl.BlockSpec((1, BE, 8), lambda i: (0, i, 0)),
          pl.BlockSpec((1, BE, 8), lambda i: (1, i, 0)),
          pl.BlockSpec((1, H), lambda i: (0, 0)),
          pl.BlockSpec((1, H), lambda i: (0, 0)),
          pl.BlockSpec((H, 2 * F), lambda i: (0, 0)),
          pl.BlockSpec((1, 2 * F), lambda i: (0, 0)),
      ],
      out_specs=pl.BlockSpec((2, BE, F), lambda i: (0, i, 0)),
      out_shape=jax.ShapeDtypeStruct((2, E, F), jnp.float32),
  )


def _gather_mul_scatter(E, N):
  U = E // CH
  tile_chunks = U // NS
  extra = U % NS
  mesh = plsc.VectorSubcoreMesh(core_axis_name="c", subcore_axis_name="s")

  scratch = [pltpu.VMEM((CH,), jnp.int32),
             pltpu.VMEM((CH,), jnp.int32),
             pltpu.VMEM((CH, F), jnp.float32),
             pltpu.VMEM((CH, F), jnp.float32),
             pltpu.VMEM((CH, F), jnp.float32),
             pltpu.VMEM((256, F), jnp.float32),
             pltpu.VMEM_SHARED((N, F), jnp.float32),
             pltpu.SemaphoreType.DMA]

  @functools.partial(
      pl.kernel,
      compiler_params=pltpu.CompilerParams(use_tc_tiling_on_sc=False),
      out_type=jax.ShapeDtypeStruct((NC * N, F), jnp.float32),
      mesh=mesh,
      scratch_types=scratch)
  def body(hsplit, rows, cols, fsplit, agg_out, *sc):
    ir, ic, hb, fb, mb, zb, agg_sh, sem = sc

    c = lax.axis_index("c")
    s = lax.axis_index("s")
    _zero_vmem(zb, 256, F)
    _tile_slab_init(zb, 256, agg_sh, s, N)
    plsc.subcore_barrier()
    cE = c * E
    cN = c * N
    n = tile_chunks + jnp.where(s < extra, 1, 0)
    base_t = s * tile_chunks + jnp.minimum(s, extra)

    def chunk_sync(k, carry):
      off = (base_t + k) * CH
      pltpu.sync_copy(rows.at[pl.ds(off, CH)], ir)
      pltpu.sync_copy(cols.at[pl.ds(off, CH)], ic)
      for g in range(CH // L):
        ic[pl.ds(g * L, L)] = ic[pl.ds(g * L, L)] + cN
      cp = pltpu.async_copy(hsplit.at[ic], hb, sem)
      pltpu.sync_copy(fsplit.at[pl.ds(cE + off, CH)], fb)
      cp.wait()
      for r in range(CH):
        for q in range(F // L):
          mb[r, pl.ds(q * L, L)] = (hb[r, pl.ds(q * L, L)]
                                    * fb[r, pl.ds(q * L, L)])
      pltpu.sync_copy(mb, agg_sh.at[ir], add=True)
      return carry

    lax.fori_loop(0, n, chunk_sync, 0)
    plsc.subcore_barrier()
    _tile_slab_drain(agg_sh, agg_out, c * N, s, N, zb, 256)

  return body


def _update_kernel(N, D, H, BN=None):
  BN = BN or _pick_block(N, 1024)

  def body(h_ref, a0_ref, a1_ref, c0_ref, c1_ref,
           w3_ref, b3_ref, g_ref, bb_ref, w4_ref, b4_ref, out_ref):
    cnt = jnp.maximum(c0_ref[0] + c1_ref[0], 1.0)
    agg = jnp.concatenate([a0_ref[0], a1_ref[0]], axis=1) / cnt
    u = jnp.concatenate([h_ref[...], agg], axis=1)
    t = jnp.dot(u, w3_ref[...],
                preferred_element_type=jnp.float32) + b3_ref[...]
    m = jnp.mean(t, axis=1, keepdims=True)
    v = jnp.mean((t - m) ** 2, axis=1, keepdims=True)
    t = (t - m) / jnp.sqrt(v + 1e-5) * g_ref[...] + bb_ref[...]
    t = t * jax.nn.sigmoid(t)
    out_ref[...] = jnp.dot(t, w4_ref[...],
                           preferred_element_type=jnp.float32) + b4_ref[...]

  return pl.pallas_call(
      body,
      grid=(N // BN,),
      in_specs=[
          pl.BlockSpec((BN, D), lambda i: (i, 0)),
          pl.BlockSpec((1, BN, F), lambda i: (0, i, 0)),
          pl.BlockSpec((1, BN, F), lambda i: (1, i, 0)),
          pl.BlockSpec((1, BN, 1), lambda i: (0, i, 0)),
          pl.BlockSpec((1, BN, 1), lambda i: (1, i, 0)),
          pl.BlockSpec((2 * D, H), lambda i: (0, 0)),
          pl.BlockSpec((1, H), lambda i: (0, 0)),
          pl.BlockSpec((1, H), lambda i: (0, 0)),
          pl.BlockSpec((1, H), lambda i: (0, 0)),
          pl.BlockSpec((H, D), lambda i: (0, 0)),
          pl.BlockSpec((1, D), lambda i: (0, 0)),
      ],
      out_specs=pl.BlockSpec((BN, D), lambda i: (i, 0)),
      out_shape=jax.ShapeDtypeStruct((N, D), jnp.float32),
  )


def kernel(x, h, W1, b1, W2, b2, W3, b3, ln_g, ln_b, W4, b4,
           edge_indices, batch_size):
  B, N, D = h.shape
  E = edge_indices.shape[1]
  H = W1.shape[1]
  assert B == 1 and D == 2 * F
  assert E % (NC * CH) == 0
  assert N % 8 == 0

  x2 = x.reshape(N, 3)
  x8 = jnp.zeros((N, 8), jnp.float32).at[:, :3].set(x2)
  h2 = h.reshape(N, D)
  hsplit = jnp.concatenate([h2[:, :F], h2[:, F:]], axis=0)
  rows = edge_indices[0]
  cols = edge_indices[1]

  xg, cnts = _edge_dist_counts(E, N)(x8, rows, cols)
  fsplit = _filt_kernel(E, H)(xg.reshape(NC, E, 8), xg.reshape(NC, E, 8),
                              W1, b1.reshape(1, H),
                              W2, b2.reshape(1, 2 * F))
  aggflat = _gather_mul_scatter(E, N)(hsplit, rows, cols,
                                      fsplit.reshape(NC * E, F))
  agg2 = aggflat.reshape(NC, N, F)
  cnt2 = cnts.reshape(NC, N, 1)
  out = _update_kernel(N, D, H)(h2, agg2, agg2, cnt2, cnt2,
                                W3, b3.reshape(1, H), ln_g.reshape(1, H),
                                ln_b.reshape(1, H), W4, b4.reshape(1, D))
  return out.reshape(B, N, D)
